# Initial kernel scaffold; baseline (speedup 1.0000x reference)
import functools

import jax
import jax.numpy as jnp
from jax import lax
from jax.experimental import pallas as pl
from jax.experimental.pallas import tpu as pltpu
from jax.experimental.pallas import tpu_sc as plsc

_NC = 2
_NS = 16
_NW = _NC * _NS

_VOCAB = 100000
_DIM = 128
_TOTAL = 4096 * 50
_PER_W = _TOTAL // _NW
_CHUNK = 100
_NCHUNK = _PER_W // _CHUNK
_NBUF = 4


def _embed_body(idx_hbm, table_hbm, out_hbm,
                idx_v, r0, r1, r2, r3, g0, g1, g2, g3, wsem):
    rows = (r0, r1, r2, r3)
    gsems = (g0, g1, g2, g3)
    wid = lax.axis_index("s") * _NC + lax.axis_index("c")
    base = wid * _PER_W

    pltpu.sync_copy(idx_hbm.at[wid], idx_v)

    def fire_gather(k, b):
        pltpu.async_copy(table_hbm.at[idx_v.at[k]], rows[b], gsems[b])

    def drain_gather(b):
        pltpu.make_async_copy(table_hbm.at[pl.ds(0, _CHUNK)], rows[b],
                              gsems[b]).wait()

    def emit_write(k, b):
        return pltpu.async_copy(
            rows[b], out_hbm.at[pl.ds(base + k * _CHUNK, _CHUNK)], wsem)

    for b in range(_NBUF):
        fire_gather(b, b)

    @pl.loop(0, _NCHUNK - _NBUF, step=_NBUF)
    def _(c):
        for b in range(_NBUF):
            k = c + b
            drain_gather(b)
            w = emit_write(k, b)
            w.wait()
            fire_gather(k + _NBUF, b)

    for b in range(_NBUF):
        k = _NCHUNK - _NBUF + b
        drain_gather(b)
        w = emit_write(k, b)
        w.wait()


@functools.partial(jax.jit, static_argnums=())
def _embed(idx, table):
    mesh = plsc.VectorSubcoreMesh(core_axis_name="c", subcore_axis_name="s")
    f = pl.kernel(
        _embed_body,
        out_type=jax.ShapeDtypeStruct((_TOTAL, _DIM), jnp.float32),
        mesh=mesh,
        scratch_types=[
            pltpu.VMEM((_NCHUNK, _CHUNK), jnp.int32),
            pltpu.VMEM((_CHUNK, _DIM), jnp.float32),
            pltpu.VMEM((_CHUNK, _DIM), jnp.float32),
            pltpu.VMEM((_CHUNK, _DIM), jnp.float32),
            pltpu.VMEM((_CHUNK, _DIM), jnp.float32),
            pltpu.SemaphoreType.DMA,
            pltpu.SemaphoreType.DMA,
            pltpu.SemaphoreType.DMA,
            pltpu.SemaphoreType.DMA,
            pltpu.SemaphoreType.DMA,
        ],
    )
    return f(idx, table)


def kernel(index, table):
    b, l = index.shape
    idx = index.astype(jnp.int32).reshape(_NW, _NCHUNK, _CHUNK)
    out = _embed(idx, table)
    return out.reshape(b, l, table.shape[1])


# speedup vs baseline: 3.3490x; 3.3490x over previous
import functools

import jax
import jax.numpy as jnp
from jax import lax
from jax.experimental import pallas as pl
from jax.experimental.pallas import tpu as pltpu
from jax.experimental.pallas import tpu_sc as plsc

_NC = 2
_NS = 16
_NW = _NC * _NS

_VOCAB = 100000
_DIM = 128
_TOTAL = 4096 * 50
_PER_W = _TOTAL // _NW
_CHUNK = 128
_NCHUNK = _PER_W // _CHUNK
_NBUF = 5


def _embed_body(idx_hbm, table_hbm, out_hbm,
                idx_v, r0, r1, r2, r3, r4, g0, g1, g2, g3, g4, wsem):
    rows = (r0, r1, r2, r3, r4)
    gsems = (g0, g1, g2, g3, g4)
    wid = lax.axis_index("s") * _NC + lax.axis_index("c")
    base = wid * _PER_W

    pltpu.sync_copy(idx_hbm.at[wid], idx_v)

    def fire_gather(k, b):
        pltpu.async_copy(table_hbm.at[idx_v.at[k]], rows[b], gsems[b])

    def drain_gather(b):
        pltpu.make_async_copy(table_hbm.at[pl.ds(0, _CHUNK)], rows[b],
                              gsems[b]).wait()

    def emit_write(k, b):
        return pltpu.async_copy(
            rows[b], out_hbm.at[pl.ds(base + k * _CHUNK, _CHUNK)], wsem)

    for b in range(_NBUF):
        fire_gather(b, b)

    @pl.loop(0, _NCHUNK - _NBUF, step=_NBUF)
    def _(c):
        for b in range(_NBUF):
            k = c + b
            drain_gather(b)
            w = emit_write(k, b)
            w.wait()
            fire_gather(k + _NBUF, b)

    for b in range(_NBUF):
        k = _NCHUNK - _NBUF + b
        drain_gather(b)
        w = emit_write(k, b)
        w.wait()


@functools.partial(jax.jit, static_argnums=())
def _embed(idx, table):
    mesh = plsc.VectorSubcoreMesh(core_axis_name="c", subcore_axis_name="s")
    f = pl.kernel(
        _embed_body,
        out_type=jax.ShapeDtypeStruct((_TOTAL, _DIM), jnp.float32),
        mesh=mesh,
        scratch_types=[
            pltpu.VMEM((_NCHUNK, _CHUNK), jnp.int32),
            pltpu.VMEM((_CHUNK, _DIM), jnp.float32),
            pltpu.VMEM((_CHUNK, _DIM), jnp.float32),
            pltpu.VMEM((_CHUNK, _DIM), jnp.float32),
            pltpu.VMEM((_CHUNK, _DIM), jnp.float32),
            pltpu.VMEM((_CHUNK, _DIM), jnp.float32),
            pltpu.SemaphoreType.DMA,
            pltpu.SemaphoreType.DMA,
            pltpu.SemaphoreType.DMA,
            pltpu.SemaphoreType.DMA,
            pltpu.SemaphoreType.DMA,
            pltpu.SemaphoreType.DMA,
        ],
    )
    return f(idx, table)


def kernel(index, table):
    b, l = index.shape
    idx = index.astype(jnp.int32).reshape(_NW, _NCHUNK, _CHUNK)
    out = _embed(idx, table)
    return out.reshape(b, l, table.shape[1])


# speedup vs baseline: 3.3504x; 1.0004x over previous
import functools

import jax
import jax.numpy as jnp
from jax import lax
from jax.experimental import pallas as pl
from jax.experimental.pallas import tpu as pltpu
from jax.experimental.pallas import tpu_sc as plsc

_NC = 2
_NS = 16
_NW = _NC * _NS

_DIM = 128
_TOTAL = 4096 * 50
_PER_W = _TOTAL // _NW
_CHUNK = 128
_NCHUNK = _PER_W // _CHUNK
_NBUF = 7


def _embed_body(idx_hbm, table_hbm, out_hbm, idx_v, *scratch):
    rows = scratch[:_NBUF]
    gsems = scratch[_NBUF:2 * _NBUF]
    wsem = scratch[2 * _NBUF]
    wid = lax.axis_index("s") * _NC + lax.axis_index("c")
    base = wid * _PER_W

    pltpu.sync_copy(idx_hbm.at[wid], idx_v)

    def fire_gather(k, b):
        pltpu.async_copy(table_hbm.at[idx_v.at[k]], rows[b], gsems[b])

    def drain_gather(b):
        pltpu.make_async_copy(table_hbm.at[pl.ds(0, _CHUNK)], rows[b],
                              gsems[b]).wait()

    def emit_write(k, b):
        return pltpu.async_copy(
            rows[b], out_hbm.at[pl.ds(base + k * _CHUNK, _CHUNK)], wsem)

    def step(k, b, refire):
        drain_gather(b)
        w = emit_write(k, b)
        w.wait()
        if refire:
            fire_gather(k + _NBUF, b)

    for b in range(_NBUF):
        fire_gather(b, b)

    n_steady = _NCHUNK - _NBUF
    n_groups = n_steady // _NBUF
    if n_groups:
        @pl.loop(0, n_groups * _NBUF, step=_NBUF)
        def _(c):
            for b in range(_NBUF):
                step(c + b, b, refire=True)

    for k in range(n_groups * _NBUF, n_steady):
        step(k, k % _NBUF, refire=True)

    for k in range(n_steady, _NCHUNK):
        step(k, k % _NBUF, refire=False)


@jax.jit
def _embed(idx, table):
    mesh = plsc.VectorSubcoreMesh(core_axis_name="c", subcore_axis_name="s")
    f = pl.kernel(
        _embed_body,
        out_type=jax.ShapeDtypeStruct((_TOTAL, _DIM), jnp.float32),
        mesh=mesh,
        scratch_types=(
            [pltpu.VMEM((_NCHUNK, _CHUNK), jnp.int32)]
            + [pltpu.VMEM((_CHUNK, _DIM), jnp.float32)] * _NBUF
            + [pltpu.SemaphoreType.DMA] * (_NBUF + 1)
        ),
    )
    return f(idx, table)


def kernel(index, table):
    b, l = index.shape
    idx = index.astype(jnp.int32).reshape(_NW, _NCHUNK, _CHUNK)
    out = _embed(idx, table)
    return out.reshape(b, l, table.shape[1])
